# Initial kernel scaffold; baseline (speedup 1.0000x reference)
import jax
import jax.numpy as jnp
from jax.experimental import pallas as pl


def kernel(x, adj, W1, b1, W2, b2, Wp, bp):
    raise NotImplementedError("write your pallas kernel here")


# speedup vs baseline: 55.4635x; 55.4635x over previous
import jax
import jax.numpy as jnp
from jax.experimental import pallas as pl
from jax.experimental.pallas import tpu as pltpu

B = 64
N = 128
F = 512
H1 = 256
H2 = 128
EMB = 32
THR = 0.62

TB = 8
KB = 2048
NB = 512


def _gcn_kernel(adj_ref, x_ref, w1_ref, b1_ref, w2_ref, b2_ref, out_ref, m_ref):
    @pl.when(pl.program_id(0) == 0)
    def _compute_m():
        a = jax.nn.sigmoid(adj_ref[...])
        rows = jax.lax.broadcasted_iota(jnp.int32, (N, N), 0)
        cols = jax.lax.broadcasted_iota(jnp.int32, (N, N), 1)
        w = (a > THR).astype(jnp.float32) + (rows == cols).astype(jnp.float32)
        deg = jnp.sum(w, axis=0)
        dinv = jax.lax.rsqrt(deg)
        m_ref[...] = w.T * (dinv[:, None] * dinv[None, :])

    m = jnp.broadcast_to(m_ref[...], (TB, N, N))
    xb = x_ref[...]
    t1 = jax.lax.dot_general(xb, w1_ref[...], (((2,), (0,)), ((), ())),
                             preferred_element_type=jnp.float32)
    agg1 = jax.lax.dot_general(m, t1, (((2,), (1,)), ((0,), (0,))),
                               preferred_element_type=jnp.float32)
    h1 = jnp.maximum(agg1 + b1_ref[0], 0.0)
    t2 = jax.lax.dot_general(h1, w2_ref[...], (((2,), (0,)), ((), ())),
                             preferred_element_type=jnp.float32)
    agg2 = jax.lax.dot_general(m, t2, (((2,), (1,)), ((0,), (0,))),
                               preferred_element_type=jnp.float32)
    out_ref[...] = jnp.maximum(agg2 + b2_ref[0], 0.0)


def _proj_kernel(a_ref, wp_ref, bp_ref, out_ref, acc_ref):
    k = pl.program_id(1)
    part = jnp.dot(a_ref[...], wp_ref[...], preferred_element_type=jnp.float32)

    @pl.when(k == 0)
    def _init():
        acc_ref[...] = part

    @pl.when(k != 0)
    def _accum():
        acc_ref[...] += part

    @pl.when(k == pl.num_programs(1) - 1)
    def _finish():
        out_ref[...] = jnp.tanh(acc_ref[...] + bp_ref[...])


def kernel(x, adj, W1, b1, W2, b2, Wp, bp):
    h = pl.pallas_call(
        _gcn_kernel,
        grid=(B // TB,),
        in_specs=[
            pl.BlockSpec((N, N), lambda i: (0, 0)),
            pl.BlockSpec((TB, N, F), lambda i: (i, 0, 0)),
            pl.BlockSpec((F, H1), lambda i: (0, 0)),
            pl.BlockSpec((1, H1), lambda i: (0, 0)),
            pl.BlockSpec((H1, H2), lambda i: (0, 0)),
            pl.BlockSpec((1, H2), lambda i: (0, 0)),
        ],
        out_specs=pl.BlockSpec((TB, N, H2), lambda i: (i, 0, 0)),
        out_shape=jax.ShapeDtypeStruct((B, N, H2), jnp.float32),
        scratch_shapes=[pltpu.VMEM((N, N), jnp.float32)],
    )(adj, x, W1, b1.reshape(1, H1), W2, b2.reshape(1, H2))

    a = h.reshape(B, N * H2)
    out = pl.pallas_call(
        _proj_kernel,
        grid=(N * EMB // NB, N * H2 // KB),
        in_specs=[
            pl.BlockSpec((B, KB), lambda n, k: (0, k)),
            pl.BlockSpec((KB, NB), lambda n, k: (k, n)),
            pl.BlockSpec((1, NB), lambda n, k: (0, n)),
        ],
        out_specs=pl.BlockSpec((B, NB), lambda n, k: (0, n)),
        out_shape=jax.ShapeDtypeStruct((B, N * EMB), jnp.float32),
        scratch_shapes=[pltpu.VMEM((B, NB), jnp.float32)],
    )(a, Wp, bp.reshape(1, N * EMB))
    return out.reshape(B, N, EMB)


# speedup vs baseline: 65.7093x; 1.1847x over previous
import jax
import jax.numpy as jnp
from jax.experimental import pallas as pl
from jax.experimental.pallas import tpu as pltpu

B = 64
N = 128
F = 512
H1 = 256
H2 = 128
EMB = 32
THR = 0.62

TB = 8
KB = 1024


def _gcn_kernel(adj_ref, x_ref, w1_ref, b1_ref, w2_ref, b2_ref, out_ref, m_ref):
    @pl.when(pl.program_id(0) == 0)
    def _compute_m():
        a = jax.nn.sigmoid(adj_ref[...])
        rows = jax.lax.broadcasted_iota(jnp.int32, (N, N), 0)
        cols = jax.lax.broadcasted_iota(jnp.int32, (N, N), 1)
        w = (a > THR).astype(jnp.float32) + (rows == cols).astype(jnp.float32)
        deg = jnp.sum(w, axis=0)
        dinv = jax.lax.rsqrt(deg)
        m_ref[...] = w.T * (dinv[:, None] * dinv[None, :])

    m = jnp.broadcast_to(m_ref[...], (TB, N, N))
    xb = x_ref[...]
    t1 = jax.lax.dot_general(xb, w1_ref[...], (((2,), (0,)), ((), ())),
                             preferred_element_type=jnp.float32)
    agg1 = jax.lax.dot_general(m, t1, (((2,), (1,)), ((0,), (0,))),
                               preferred_element_type=jnp.float32)
    h1 = jnp.maximum(agg1 + b1_ref[0], 0.0)
    t2 = jax.lax.dot_general(h1, w2_ref[...], (((2,), (0,)), ((), ())),
                             preferred_element_type=jnp.float32)
    agg2 = jax.lax.dot_general(m, t2, (((2,), (1,)), ((0,), (0,))),
                               preferred_element_type=jnp.float32)
    h2 = jnp.maximum(agg2 + b2_ref[0], 0.0)
    out_ref[...] = h2.reshape(TB, N * H2)


def _proj_kernel(a_ref, wp_ref, bp_ref, out_ref, acc_ref):
    k = pl.program_id(0)
    part = jnp.dot(a_ref[...], wp_ref[...], preferred_element_type=jnp.float32)

    @pl.when(k == 0)
    def _init():
        acc_ref[...] = part

    @pl.when(k != 0)
    def _accum():
        acc_ref[...] += part

    @pl.when(k == pl.num_programs(0) - 1)
    def _finish():
        out_ref[...] = jnp.tanh(acc_ref[...] + bp_ref[...])


def kernel(x, adj, W1, b1, W2, b2, Wp, bp):
    h = pl.pallas_call(
        _gcn_kernel,
        grid=(B // TB,),
        in_specs=[
            pl.BlockSpec((N, N), lambda i: (0, 0)),
            pl.BlockSpec((TB, N, F), lambda i: (i, 0, 0)),
            pl.BlockSpec((F, H1), lambda i: (0, 0)),
            pl.BlockSpec((1, H1), lambda i: (0, 0)),
            pl.BlockSpec((H1, H2), lambda i: (0, 0)),
            pl.BlockSpec((1, H2), lambda i: (0, 0)),
        ],
        out_specs=pl.BlockSpec((TB, N * H2), lambda i: (i, 0)),
        out_shape=jax.ShapeDtypeStruct((B, N * H2), jnp.float32),
        scratch_shapes=[pltpu.VMEM((N, N), jnp.float32)],
    )(adj, x, W1, b1.reshape(1, H1), W2, b2.reshape(1, H2))

    out = pl.pallas_call(
        _proj_kernel,
        grid=(N * H2 // KB,),
        in_specs=[
            pl.BlockSpec((B, KB), lambda k: (0, k)),
            pl.BlockSpec((KB, N * EMB), lambda k: (k, 0)),
            pl.BlockSpec((1, N * EMB), lambda k: (0, 0)),
        ],
        out_specs=pl.BlockSpec((B, N * EMB), lambda k: (0, 0)),
        out_shape=jax.ShapeDtypeStruct((B, N * EMB), jnp.float32),
        scratch_shapes=[pltpu.VMEM((B, N * EMB), jnp.float32)],
    )(h, Wp, bp.reshape(1, N * EMB))
    return out.reshape(B, N, EMB)


# speedup vs baseline: 67.6862x; 1.0301x over previous
import jax
import jax.numpy as jnp
from jax.experimental import pallas as pl
from jax.experimental.pallas import tpu as pltpu

B = 64
N = 128
F = 512
H1 = 256
H2 = 128
EMB = 32
THR = 0.62

TB = 16
KB = 512
NK = N * H2 // KB
NBUF = 3
NPB = KB // H2


def _fused_kernel(adj_ref, x_hbm, w1_ref, b1_ref, w2_ref, b2_ref, wp_hbm,
                  bp_ref, out_ref, x_ref, h_ref, acc_ref, wbuf_ref,
                  wp_sem, x_sem):
    for i in range(NBUF):
        pltpu.make_async_copy(wp_hbm.at[pl.ds(i * KB, KB), :],
                              wbuf_ref.at[i], wp_sem.at[i]).start()
    x_copy = pltpu.make_async_copy(x_hbm, x_ref, x_sem)
    x_copy.start()

    a = jax.nn.sigmoid(adj_ref[...])
    rows = jax.lax.broadcasted_iota(jnp.int32, (N, N), 0)
    cols = jax.lax.broadcasted_iota(jnp.int32, (N, N), 1)
    w = (a > THR).astype(jnp.float32) + (rows == cols).astype(jnp.float32)
    deg = jnp.sum(w, axis=0)
    dinv = jax.lax.rsqrt(deg)
    m = w.T * (dinv[:, None] * dinv[None, :])
    mb = jnp.broadcast_to(m, (TB, N, N))

    x_copy.wait()
    for i in range(B // TB):
        xb = x_ref[i * TB:(i + 1) * TB]
        t1 = jax.lax.dot_general(xb, w1_ref[...], (((2,), (0,)), ((), ())),
                                 preferred_element_type=jnp.float32)
        agg1 = jax.lax.dot_general(mb, t1, (((2,), (1,)), ((0,), (0,))),
                                   preferred_element_type=jnp.float32)
        h1 = jnp.maximum(agg1 + b1_ref[0], 0.0)
        t2 = jax.lax.dot_general(h1, w2_ref[...], (((2,), (0,)), ((), ())),
                                 preferred_element_type=jnp.float32)
        agg2 = jax.lax.dot_general(mb, t2, (((2,), (1,)), ((0,), (0,))),
                                   preferred_element_type=jnp.float32)
        h_ref[i * TB:(i + 1) * TB] = jnp.maximum(agg2 + b2_ref[0], 0.0)

    acc_ref[...] = jnp.zeros((B, N * EMB), jnp.float32)

    def body(k, carry):
        buf = jax.lax.rem(k, NBUF)
        pltpu.make_async_copy(wp_hbm.at[pl.ds(k * KB, KB), :],
                              wbuf_ref.at[buf], wp_sem.at[buf]).wait()
        wblk = wbuf_ref[buf]
        hblk = h_ref[:, pl.ds(k * NPB, NPB), :]
        part = acc_ref[...]
        for c in range(NPB):
            part = part + jnp.dot(hblk[:, c, :],
                                  wblk[c * H2:(c + 1) * H2, :],
                                  preferred_element_type=jnp.float32)
        acc_ref[...] = part

        @pl.when(k + NBUF < NK)
        def _():
            pltpu.make_async_copy(wp_hbm.at[pl.ds((k + NBUF) * KB, KB), :],
                                  wbuf_ref.at[buf], wp_sem.at[buf]).start()
        return carry

    jax.lax.fori_loop(0, NK, body, 0)
    out_ref[...] = jnp.tanh(acc_ref[...] + bp_ref[...])


def kernel(x, adj, W1, b1, W2, b2, Wp, bp):
    out = pl.pallas_call(
        _fused_kernel,
        in_specs=[
            pl.BlockSpec(memory_space=pltpu.MemorySpace.VMEM),
            pl.BlockSpec(memory_space=pl.ANY),
            pl.BlockSpec(memory_space=pltpu.MemorySpace.VMEM),
            pl.BlockSpec(memory_space=pltpu.MemorySpace.VMEM),
            pl.BlockSpec(memory_space=pltpu.MemorySpace.VMEM),
            pl.BlockSpec(memory_space=pltpu.MemorySpace.VMEM),
            pl.BlockSpec(memory_space=pl.ANY),
            pl.BlockSpec(memory_space=pltpu.MemorySpace.VMEM),
        ],
        out_specs=pl.BlockSpec(memory_space=pltpu.MemorySpace.VMEM),
        out_shape=jax.ShapeDtypeStruct((B, N * EMB), jnp.float32),
        scratch_shapes=[
            pltpu.VMEM((B, N, F), jnp.float32),
            pltpu.VMEM((B, N, H2), jnp.float32),
            pltpu.VMEM((B, N * EMB), jnp.float32),
            pltpu.VMEM((NBUF, KB, N * EMB), jnp.float32),
            pltpu.SemaphoreType.DMA((NBUF,)),
            pltpu.SemaphoreType.DMA,
        ],
    )(adj, x, W1, b1.reshape(1, H1), W2, b2.reshape(1, H2), Wp,
      bp.reshape(1, N * EMB))
    return out.reshape(B, N, EMB)
